# Initial kernel scaffold; baseline (speedup 1.0000x reference)
import jax
import jax.numpy as jnp
from jax.experimental import pallas as pl


def kernel(point_seq):
    raise NotImplementedError("write your pallas kernel here")


# speedup vs baseline: 3.3280x; 3.3280x over previous
import functools

import jax
import jax.numpy as jnp
from jax import lax
from jax.experimental import pallas as pl

K = 16
N = 2048
QB = 256


def _knn_body(x2_ref, x1t_ref, idx_ref, dist_ref):
    qx = x2_ref[0, :, 0:1]
    qy = x2_ref[0, :, 1:2]
    qz = x2_ref[0, :, 2:3]
    bx = x1t_ref[0, 0:1, :]
    by = x1t_ref[0, 1:2, :]
    bz = x1t_ref[0, 2:3, :]
    qn = qx * qx + qy * qy + qz * qz
    bn = bx * bx + by * by + bz * bz
    qxb = qx.astype(jnp.bfloat16).astype(jnp.float32)
    qyb = qy.astype(jnp.bfloat16).astype(jnp.float32)
    qzb = qz.astype(jnp.bfloat16).astype(jnp.float32)
    bxb = bx.astype(jnp.bfloat16).astype(jnp.float32)
    byb = by.astype(jnp.bfloat16).astype(jnp.float32)
    bzb = bz.astype(jnp.bfloat16).astype(jnp.float32)
    dot = qxb * bxb + qyb * byb + qzb * bzb
    d2 = (qn - 2.0 * dot) + bn
    col = lax.broadcasted_iota(jnp.int32, (QB, N), 1)
    inf = jnp.float32(jnp.inf)
    idx_cols = []
    dist_cols = []
    for _ in range(K):
        m = jnp.min(d2, axis=1, keepdims=True)
        sel = jnp.min(jnp.where(d2 == m, col, N), axis=1, keepdims=True)
        idx_cols.append(sel)
        dist_cols.append(jnp.sqrt(jnp.maximum(m, 0.0)))
        d2 = jnp.where(col == sel, inf, d2)
    idx_ref[0] = jnp.concatenate(idx_cols, axis=1)
    dist_ref[0] = jnp.concatenate(dist_cols, axis=1)


def _knn(x2, x1t, interpret=False):
    bt = x2.shape[0]
    return pl.pallas_call(
        _knn_body,
        grid=(bt, N // QB),
        in_specs=[
            pl.BlockSpec((1, QB, 3), lambda i, j: (i, j, 0)),
            pl.BlockSpec((1, 3, N), lambda i, j: (i, 0, 0)),
        ],
        out_specs=[
            pl.BlockSpec((1, QB, K), lambda i, j: (i, j, 0)),
            pl.BlockSpec((1, QB, K), lambda i, j: (i, j, 0)),
        ],
        out_shape=[
            jax.ShapeDtypeStruct((bt, N, K), jnp.int32),
            jax.ShapeDtypeStruct((bt, N, K), jnp.float32),
        ],
        interpret=interpret,
    )(x2, x1t)


def kernel(point_seq):
    b, t, n, d = point_seq.shape
    x1 = point_seq.reshape(-1, n, d)
    x2 = jnp.concatenate([point_seq[:, :1], point_seq], axis=1)[:, :-1]
    x2 = x2.reshape(-1, n, d)
    bt = x1.shape[0]

    idxs, dists = _knn(x2, jnp.transpose(x1, (0, 2, 1)))

    def step(prev, idx_i):
        pat = idx_i[prev, :]
        return pat[:, 0], pat

    _, pats = lax.scan(step, jnp.arange(n, dtype=jnp.int32), idxs)
    patchlet_points = jax.vmap(lambda pts, idx: pts[idx])(x1, pats)
    anchor = patchlet_points.reshape(b, t, n, K, d)[:, 0, :, 0:1, :][:, None]
    norm = patchlet_points.reshape(b, t, n, K, d) - anchor
    feats = jnp.concatenate([patchlet_points.reshape(b, t, n, K, d), norm], -1)

    return (
        idxs.reshape(b, t, n, K),
        dists.reshape(b, t, n, K),
        pats.reshape(b, t, n, K),
        patchlet_points.reshape(b, t, n, K, d),
        feats,
        norm,
    )


# speedup vs baseline: 13.7812x; 4.1409x over previous
import functools

import jax
import jax.numpy as jnp
from jax import lax
from jax.experimental import pallas as pl
from jax.experimental.pallas import tpu as pltpu
from jax.experimental.pallas import tpu_sc as plsc

K = 16
N = 2048
QB = 256
BT = 64
NW = 32
QPW = N // NW


def _knn_body(x2_ref, x1t_ref, idx_ref, dist_ref):
    qx = x2_ref[0, :, 0:1]
    qy = x2_ref[0, :, 1:2]
    qz = x2_ref[0, :, 2:3]
    bx = x1t_ref[0, 0:1, :]
    by = x1t_ref[0, 1:2, :]
    bz = x1t_ref[0, 2:3, :]
    qn = qx * qx + qy * qy + qz * qz
    bn = bx * bx + by * by + bz * bz
    qxb = qx.astype(jnp.bfloat16).astype(jnp.float32)
    qyb = qy.astype(jnp.bfloat16).astype(jnp.float32)
    qzb = qz.astype(jnp.bfloat16).astype(jnp.float32)
    bxb = bx.astype(jnp.bfloat16).astype(jnp.float32)
    byb = by.astype(jnp.bfloat16).astype(jnp.float32)
    bzb = bz.astype(jnp.bfloat16).astype(jnp.float32)
    dot = qxb * bxb + qyb * byb + qzb * bzb
    d2 = (qn - 2.0 * dot) + bn
    col = lax.broadcasted_iota(jnp.int32, (QB, N), 1)
    inf = jnp.float32(jnp.inf)
    idx_cols = []
    dist_cols = []
    for _ in range(K):
        m = jnp.min(d2, axis=1, keepdims=True)
        sel = jnp.min(jnp.where(d2 == m, col, N), axis=1, keepdims=True)
        idx_cols.append(sel)
        dist_cols.append(jnp.sqrt(jnp.maximum(m, 0.0)))
        d2 = jnp.where(col == sel, inf, d2)
    idx_ref[0] = jnp.concatenate(idx_cols, axis=1)
    dist_ref[0] = jnp.concatenate(dist_cols, axis=1)


def _knn(x2, x1t, interpret=False):
    bt = x2.shape[0]
    return pl.pallas_call(
        _knn_body,
        grid=(bt, N // QB),
        in_specs=[
            pl.BlockSpec((1, QB, 3), lambda i, j: (i, j, 0)),
            pl.BlockSpec((1, 3, N), lambda i, j: (i, 0, 0)),
        ],
        out_specs=[
            pl.BlockSpec((1, QB, K), lambda i, j: (i, j, 0)),
            pl.BlockSpec((1, QB, K), lambda i, j: (i, j, 0)),
        ],
        out_shape=[
            jax.ShapeDtypeStruct((bt, N, K), jnp.int32),
            jax.ShapeDtypeStruct((bt, N, K), jnp.float32),
        ],
        interpret=interpret,
    )(x2, x1t)


def _chain_body(idx_hbm, x1_hbm, pat_hbm, pts_hbm, feats_hbm, norm_hbm,
                prev_v, pat_v, patf_v, x1_v, pts_v, feats_v, norm_v, anch_v,
                sem):
    cid = lax.axis_index("c")
    sid = lax.axis_index("s")
    base = (sid * 2 + cid) * QPW
    lane = lax.iota(jnp.int32, 16)
    for c in range(QPW // 16):
        prev_v[pl.ds(c * 16, 16)] = base + c * 16 + lane

    def frame_body(i, carry):
        pltpu.sync_copy(x1_hbm.at[pl.ds(i * N * 3, N * 3)], x1_v)
        pltpu.async_copy(idx_hbm.at[prev_v], pat_v, sem).wait()
        nextoff = (i + 1) * N
        for q in range(QPW):
            rows = pat_v[q, :]
            patf_v[pl.ds(q * K, 16)] = rows
            xv = plsc.load_gather(x1_v, [rows * 3])
            yv = plsc.load_gather(x1_v, [rows * 3 + 1])
            zv = plsc.load_gather(x1_v, [rows * 3 + 2])

            @pl.when(i % 16 == 0)
            def _():
                l0 = lane == 0
                zf = jnp.float32(0.0)
                anch_v[pl.ds(q * 48, 16)] = plsc.cumsum(
                    jnp.where(l0, xv, zf))
                anch_v[pl.ds(q * 48 + 16, 16)] = plsc.cumsum(
                    jnp.where(l0, yv, zf))
                anch_v[pl.ds(q * 48 + 32, 16)] = plsc.cumsum(
                    jnp.where(l0, zv, zf))

            ax = anch_v[pl.ds(q * 48, 16)]
            ay = anch_v[pl.ds(q * 48 + 16, 16)]
            az = anch_v[pl.ds(q * 48 + 32, 16)]
            nx, ny, nz = xv - ax, yv - ay, zv - az
            p3 = q * (K * 3) + lane * 3
            p6 = q * (K * 6) + lane * 6
            plsc.store_scatter(pts_v, [p3], xv)
            plsc.store_scatter(pts_v, [p3 + 1], yv)
            plsc.store_scatter(pts_v, [p3 + 2], zv)
            plsc.store_scatter(norm_v, [p3], nx)
            plsc.store_scatter(norm_v, [p3 + 1], ny)
            plsc.store_scatter(norm_v, [p3 + 2], nz)
            plsc.store_scatter(feats_v, [p6], xv)
            plsc.store_scatter(feats_v, [p6 + 1], yv)
            plsc.store_scatter(feats_v, [p6 + 2], zv)
            plsc.store_scatter(feats_v, [p6 + 3], nx)
            plsc.store_scatter(feats_v, [p6 + 4], ny)
            plsc.store_scatter(feats_v, [p6 + 5], nz)
            plsc.store_scatter(prev_v, [jnp.full((16,), q, jnp.int32)],
                               rows + nextoff, mask=lane == 0)
        pltpu.sync_copy(patf_v, pat_hbm.at[pl.ds((i * N + base) * K, QPW * K)])
        pltpu.sync_copy(pts_v,
                        pts_hbm.at[pl.ds((i * N + base) * K * 3, QPW * K * 3)])
        pltpu.sync_copy(feats_v,
                        feats_hbm.at[pl.ds((i * N + base) * K * 6, QPW * K * 6)])
        pltpu.sync_copy(norm_v,
                        norm_hbm.at[pl.ds((i * N + base) * K * 3, QPW * K * 3)])
        return carry

    lax.fori_loop(0, BT, frame_body, 0)


def _chain(idx_flat, x1_flat):
    f = pl.kernel(
        _chain_body,
        mesh=plsc.VectorSubcoreMesh(core_axis_name="c", subcore_axis_name="s"),
        compiler_params=pltpu.CompilerParams(
            needs_layout_passes=False, use_tc_tiling_on_sc=False),
        out_type=[
            jax.ShapeDtypeStruct((BT * N * K,), jnp.int32),
            jax.ShapeDtypeStruct((BT * N * K * 3,), jnp.float32),
            jax.ShapeDtypeStruct((BT * N * K * 6,), jnp.float32),
            jax.ShapeDtypeStruct((BT * N * K * 3,), jnp.float32),
        ],
        scratch_types=[
            pltpu.VMEM((QPW,), jnp.int32),
            pltpu.VMEM((QPW, K), jnp.int32),
            pltpu.VMEM((QPW * K,), jnp.int32),
            pltpu.VMEM((N * 3,), jnp.float32),
            pltpu.VMEM((QPW * K * 3,), jnp.float32),
            pltpu.VMEM((QPW * K * 6,), jnp.float32),
            pltpu.VMEM((QPW * K * 3,), jnp.float32),
            pltpu.VMEM((QPW * 48,), jnp.float32),
            pltpu.SemaphoreType.DMA,
        ],
    )
    return f(idx_flat, x1_flat)


def kernel(point_seq):
    b, t, n, d = point_seq.shape
    x1 = point_seq.reshape(-1, n, d)
    x2 = jnp.concatenate([point_seq[:, :1], point_seq], axis=1)[:, :-1]
    x2 = x2.reshape(-1, n, d)

    idxs, dists = _knn(x2, jnp.transpose(x1, (0, 2, 1)))

    pats, patchlet_points, feats, norm = _chain(
        idxs.reshape(BT * N, K), x1.reshape(-1))

    return (
        idxs.reshape(b, t, n, K),
        dists.reshape(b, t, n, K),
        pats.reshape(b, t, n, K),
        patchlet_points.reshape(b, t, n, K, d),
        feats.reshape(b, t, n, K, 2 * d),
        norm.reshape(b, t, n, K, d),
    )


# speedup vs baseline: 18.5940x; 1.3492x over previous
import functools

import jax
import jax.numpy as jnp
from jax import lax
from jax.experimental import pallas as pl
from jax.experimental.pallas import tpu as pltpu
from jax.experimental.pallas import tpu_sc as plsc

K = 16
N = 2048
QB = 256
BT = 64
NW = 32
QPW = N // NW


def _knn_body(x1t_ref, x1_ref, idx_ref, dist_ref):
    qx = x1t_ref[0, 0:1, :]
    qy = x1t_ref[0, 1:2, :]
    qz = x1t_ref[0, 2:3, :]
    bx = x1_ref[0, :, 0:1]
    by = x1_ref[0, :, 1:2]
    bz = x1_ref[0, :, 2:3]
    qn = qx * qx + qy * qy + qz * qz
    bn = bx * bx + by * by + bz * bz
    qxb = qx.astype(jnp.bfloat16).astype(jnp.float32)
    qyb = qy.astype(jnp.bfloat16).astype(jnp.float32)
    qzb = qz.astype(jnp.bfloat16).astype(jnp.float32)
    bxb = bx.astype(jnp.bfloat16).astype(jnp.float32)
    byb = by.astype(jnp.bfloat16).astype(jnp.float32)
    bzb = bz.astype(jnp.bfloat16).astype(jnp.float32)
    dot = qxb * bxb + qyb * byb + qzb * bzb
    d2 = (qn - 2.0 * dot) + bn
    row = lax.broadcasted_iota(jnp.int32, (N, QB), 0)
    inf = jnp.float32(jnp.inf)
    idx_rows = []
    dist_rows = []
    for _ in range(K):
        m = jnp.min(d2, axis=0, keepdims=True)
        sel = jnp.min(jnp.where(d2 == m, row, N), axis=0, keepdims=True)
        idx_rows.append(sel)
        dist_rows.append(jnp.sqrt(jnp.maximum(m, 0.0)))
        d2 = jnp.where(row == sel, inf, d2)
    idx_ref[0] = jnp.concatenate(idx_rows, axis=0)
    dist_ref[0] = jnp.concatenate(dist_rows, axis=0)


def _knn(x1t, x1, interpret=False):
    def qmap(i, j):
        return (jnp.where(i % 16 == 0, i, i - 1), 0, j)

    return pl.pallas_call(
        _knn_body,
        grid=(BT, N // QB),
        in_specs=[
            pl.BlockSpec((1, 3, QB), qmap),
            pl.BlockSpec((1, N, 3), lambda i, j: (i, 0, 0)),
        ],
        out_specs=[
            pl.BlockSpec((1, K, QB), lambda i, j: (i, 0, j)),
            pl.BlockSpec((1, K, QB), lambda i, j: (i, 0, j)),
        ],
        out_shape=[
            jax.ShapeDtypeStruct((BT, K, N), jnp.int32),
            jax.ShapeDtypeStruct((BT, K, N), jnp.float32),
        ],
        interpret=interpret,
    )(x1t, x1)


def _chain_body(idxt_hbm, x1_hbm, pat_hbm, pts_hbm, feats_hbm, norm_hbm,
                prevb_v, idxf_v, x1_v, patf_v, pts_v, feats_v, norm_v,
                anch_v):
    cid = lax.axis_index("c")
    sid = lax.axis_index("s")
    base = (sid * 2 + cid) * QPW
    lane = lax.iota(jnp.int32, 16)
    for q in range(QPW):
        prevb_v[pl.ds(q * 16, 16)] = jnp.full((16,), base + q, jnp.int32)

    def frame_body(i, carry):
        pltpu.sync_copy(idxt_hbm.at[pl.ds(i * K * N, K * N)], idxf_v)
        pltpu.sync_copy(x1_hbm.at[pl.ds(i * N * 3, N * 3)], x1_v)
        for q in range(QPW):
            pv = prevb_v[pl.ds(q * 16, 16)]
            rows = plsc.load_gather(idxf_v, [lane * N + pv])
            q16 = jnp.full((16,), q, jnp.int32)
            plsc.store_scatter(patf_v, [lane, q16], rows)
            xv = plsc.load_gather(x1_v, [rows * 3])
            yv = plsc.load_gather(x1_v, [rows * 3 + 1])
            zv = plsc.load_gather(x1_v, [rows * 3 + 2])

            @pl.when(i % 16 == 0)
            def _():
                l0 = lane == 0
                zf = jnp.float32(0.0)
                anch_v[pl.ds(q * 48, 16)] = plsc.cumsum(
                    jnp.where(l0, xv, zf))
                anch_v[pl.ds(q * 48 + 16, 16)] = plsc.cumsum(
                    jnp.where(l0, yv, zf))
                anch_v[pl.ds(q * 48 + 32, 16)] = plsc.cumsum(
                    jnp.where(l0, zv, zf))

            ax = anch_v[pl.ds(q * 48, 16)]
            ay = anch_v[pl.ds(q * 48 + 16, 16)]
            az = anch_v[pl.ds(q * 48 + 32, 16)]
            nx, ny, nz = xv - ax, yv - ay, zv - az
            z16 = jnp.zeros((16,), jnp.int32)
            plsc.store_scatter(pts_v, [z16, lane, q16], xv)
            plsc.store_scatter(pts_v, [z16 + 1, lane, q16], yv)
            plsc.store_scatter(pts_v, [z16 + 2, lane, q16], zv)
            plsc.store_scatter(norm_v, [z16, lane, q16], nx)
            plsc.store_scatter(norm_v, [z16 + 1, lane, q16], ny)
            plsc.store_scatter(norm_v, [z16 + 2, lane, q16], nz)
            plsc.store_scatter(feats_v, [z16, lane, q16], xv)
            plsc.store_scatter(feats_v, [z16 + 1, lane, q16], yv)
            plsc.store_scatter(feats_v, [z16 + 2, lane, q16], zv)
            plsc.store_scatter(feats_v, [z16 + 3, lane, q16], nx)
            plsc.store_scatter(feats_v, [z16 + 4, lane, q16], ny)
            plsc.store_scatter(feats_v, [z16 + 5, lane, q16], nz)
            prevb_v[pl.ds(q * 16, 16)] = plsc.cumsum(
                jnp.where(lane == 0, rows, 0))
        pltpu.sync_copy(patf_v, pat_hbm.at[i, :, pl.ds(base, QPW)])
        pltpu.sync_copy(pts_v, pts_hbm.at[i, :, :, pl.ds(base, QPW)])
        pltpu.sync_copy(feats_v, feats_hbm.at[i, :, :, pl.ds(base, QPW)])
        pltpu.sync_copy(norm_v, norm_hbm.at[i, :, :, pl.ds(base, QPW)])
        return carry

    lax.fori_loop(0, BT, frame_body, 0)


def _chain(idxt_flat, x1_flat):
    f = pl.kernel(
        _chain_body,
        mesh=plsc.VectorSubcoreMesh(core_axis_name="c", subcore_axis_name="s"),
        compiler_params=pltpu.CompilerParams(
            needs_layout_passes=False, use_tc_tiling_on_sc=False),
        out_type=[
            jax.ShapeDtypeStruct((BT, K, N), jnp.int32),
            jax.ShapeDtypeStruct((BT, 3, K, N), jnp.float32),
            jax.ShapeDtypeStruct((BT, 6, K, N), jnp.float32),
            jax.ShapeDtypeStruct((BT, 3, K, N), jnp.float32),
        ],
        scratch_types=[
            pltpu.VMEM((QPW * 16,), jnp.int32),
            pltpu.VMEM((K * N,), jnp.int32),
            pltpu.VMEM((N * 3,), jnp.float32),
            pltpu.VMEM((K, QPW), jnp.int32),
            pltpu.VMEM((3, K, QPW), jnp.float32),
            pltpu.VMEM((6, K, QPW), jnp.float32),
            pltpu.VMEM((3, K, QPW), jnp.float32),
            pltpu.VMEM((QPW * 48,), jnp.float32),
        ],
    )
    return f(idxt_flat, x1_flat)


def kernel(point_seq):
    b, t, n, d = point_seq.shape
    x1 = point_seq.reshape(-1, n, d)
    x1t = jnp.transpose(x1, (0, 2, 1))

    idxs_t, dists_t = _knn(x1t, x1)

    pats_t, pts_t, feats_t, norm_t = _chain(
        idxs_t.reshape(-1), x1.reshape(-1))

    def unt(a):
        return a.reshape(b, t, K, n).transpose(0, 1, 3, 2)

    def untc(a, c):
        return a.reshape(b, t, c, K, n).transpose(0, 1, 4, 3, 2)

    return (
        unt(idxs_t),
        unt(dists_t),
        unt(pats_t),
        untc(pts_t, 3),
        untc(feats_t, 6),
        untc(norm_t, 3),
    )


# speedup vs baseline: 21.5392x; 1.1584x over previous
import functools

import jax
import jax.numpy as jnp
from jax import lax
from jax.experimental import pallas as pl
from jax.experimental.pallas import tpu as pltpu
from jax.experimental.pallas import tpu_sc as plsc

K = 16
N = 2048
QB = 512
BT = 64
NW = 32
QPW = N // NW


def _knn_body(x1t_ref, x1_ref, idx_ref, dist_ref):
    qx = x1t_ref[0, 0:1, :]
    qy = x1t_ref[0, 1:2, :]
    qz = x1t_ref[0, 2:3, :]
    bx = x1_ref[0, :, 0:1]
    by = x1_ref[0, :, 1:2]
    bz = x1_ref[0, :, 2:3]
    qn = qx * qx + qy * qy + qz * qz
    bn = bx * bx + by * by + bz * bz
    qxb = qx.astype(jnp.bfloat16).astype(jnp.float32)
    qyb = qy.astype(jnp.bfloat16).astype(jnp.float32)
    qzb = qz.astype(jnp.bfloat16).astype(jnp.float32)
    bxb = bx.astype(jnp.bfloat16).astype(jnp.float32)
    byb = by.astype(jnp.bfloat16).astype(jnp.float32)
    bzb = bz.astype(jnp.bfloat16).astype(jnp.float32)
    dot = qxb * bxb + qyb * byb + qzb * bzb
    d2 = (qn - 2.0 * dot) + bn
    row = lax.broadcasted_iota(jnp.int32, (N, QB), 0)
    inf = jnp.float32(jnp.inf)
    idx_rows = []
    dist_rows = []
    for _ in range(K):
        m = jnp.min(d2, axis=0, keepdims=True)
        sel = jnp.min(jnp.where(d2 == m, row, N), axis=0, keepdims=True)
        idx_rows.append(sel)
        dist_rows.append(jnp.sqrt(jnp.maximum(m, 0.0)))
        d2 = jnp.where(row == sel, inf, d2)
    idx_ref[0] = jnp.concatenate(idx_rows, axis=0)
    dist_ref[0] = jnp.concatenate(dist_rows, axis=0)


def _knn(x1t, x1, interpret=False):
    def qmap(i, j):
        return (jnp.where(i % 16 == 0, i, i - 1), 0, j)

    return pl.pallas_call(
        _knn_body,
        grid=(BT, N // QB),
        in_specs=[
            pl.BlockSpec((1, 3, QB), qmap),
            pl.BlockSpec((1, N, 3), lambda i, j: (i, 0, 0)),
        ],
        out_specs=[
            pl.BlockSpec((1, K, QB), lambda i, j: (i, 0, j)),
            pl.BlockSpec((1, K, QB), lambda i, j: (i, 0, j)),
        ],
        out_shape=[
            jax.ShapeDtypeStruct((BT, K, N), jnp.int32),
            jax.ShapeDtypeStruct((BT, K, N), jnp.float32),
        ],
        interpret=interpret,
    )(x1t, x1)


def _chain_body(idxt_hbm, x1_hbm, pat_hbm, pts_hbm, feats_hbm, norm_hbm,
                prevb_v, idxf_v, x1_v, patf_v, pts_v, feats_v, norm_v,
                anch_v):
    cid = lax.axis_index("c")
    sid = lax.axis_index("s")
    base = (sid * 2 + cid) * QPW
    lane = lax.iota(jnp.int32, 16)
    for q in range(QPW):
        prevb_v[pl.ds(q * 16, 16)] = jnp.full((16,), base + q, jnp.int32)

    def frame_body(i, carry):
        pltpu.sync_copy(idxt_hbm.at[pl.ds(i * K * N, K * N)], idxf_v)
        pltpu.sync_copy(x1_hbm.at[pl.ds(i * N * 3, N * 3)], x1_v)
        for q in range(QPW):
            pv = prevb_v[pl.ds(q * 16, 16)]
            rows = plsc.load_gather(idxf_v, [lane * N + pv])
            q16 = jnp.full((16,), q, jnp.int32)
            plsc.store_scatter(patf_v, [lane, q16], rows)
            xv = plsc.load_gather(x1_v, [rows * 3])
            yv = plsc.load_gather(x1_v, [rows * 3 + 1])
            zv = plsc.load_gather(x1_v, [rows * 3 + 2])

            @pl.when(i % 16 == 0)
            def _():
                l0 = lane == 0
                zf = jnp.float32(0.0)
                anch_v[pl.ds(q * 48, 16)] = plsc.cumsum(
                    jnp.where(l0, xv, zf))
                anch_v[pl.ds(q * 48 + 16, 16)] = plsc.cumsum(
                    jnp.where(l0, yv, zf))
                anch_v[pl.ds(q * 48 + 32, 16)] = plsc.cumsum(
                    jnp.where(l0, zv, zf))

            ax = anch_v[pl.ds(q * 48, 16)]
            ay = anch_v[pl.ds(q * 48 + 16, 16)]
            az = anch_v[pl.ds(q * 48 + 32, 16)]
            nx, ny, nz = xv - ax, yv - ay, zv - az
            z16 = jnp.zeros((16,), jnp.int32)
            plsc.store_scatter(pts_v, [z16, lane, q16], xv)
            plsc.store_scatter(pts_v, [z16 + 1, lane, q16], yv)
            plsc.store_scatter(pts_v, [z16 + 2, lane, q16], zv)
            plsc.store_scatter(norm_v, [z16, lane, q16], nx)
            plsc.store_scatter(norm_v, [z16 + 1, lane, q16], ny)
            plsc.store_scatter(norm_v, [z16 + 2, lane, q16], nz)
            plsc.store_scatter(feats_v, [z16, lane, q16], xv)
            plsc.store_scatter(feats_v, [z16 + 1, lane, q16], yv)
            plsc.store_scatter(feats_v, [z16 + 2, lane, q16], zv)
            plsc.store_scatter(feats_v, [z16 + 3, lane, q16], nx)
            plsc.store_scatter(feats_v, [z16 + 4, lane, q16], ny)
            plsc.store_scatter(feats_v, [z16 + 5, lane, q16], nz)
            prevb_v[pl.ds(q * 16, 16)] = plsc.cumsum(
                jnp.where(lane == 0, rows, 0))
        pltpu.sync_copy(patf_v, pat_hbm.at[i, :, pl.ds(base, QPW)])
        pltpu.sync_copy(pts_v, pts_hbm.at[i, :, :, pl.ds(base, QPW)])
        pltpu.sync_copy(feats_v, feats_hbm.at[i, :, :, pl.ds(base, QPW)])
        pltpu.sync_copy(norm_v, norm_hbm.at[i, :, :, pl.ds(base, QPW)])
        return carry

    lax.fori_loop(0, BT, frame_body, 0)


def _chain(idxt_flat, x1_flat):
    f = pl.kernel(
        _chain_body,
        mesh=plsc.VectorSubcoreMesh(core_axis_name="c", subcore_axis_name="s"),
        compiler_params=pltpu.CompilerParams(
            needs_layout_passes=False, use_tc_tiling_on_sc=False),
        out_type=[
            jax.ShapeDtypeStruct((BT, K, N), jnp.int32),
            jax.ShapeDtypeStruct((BT, 3, K, N), jnp.float32),
            jax.ShapeDtypeStruct((BT, 6, K, N), jnp.float32),
            jax.ShapeDtypeStruct((BT, 3, K, N), jnp.float32),
        ],
        scratch_types=[
            pltpu.VMEM((QPW * 16,), jnp.int32),
            pltpu.VMEM((K * N,), jnp.int32),
            pltpu.VMEM((N * 3,), jnp.float32),
            pltpu.VMEM((K, QPW), jnp.int32),
            pltpu.VMEM((3, K, QPW), jnp.float32),
            pltpu.VMEM((6, K, QPW), jnp.float32),
            pltpu.VMEM((3, K, QPW), jnp.float32),
            pltpu.VMEM((QPW * 48,), jnp.float32),
        ],
    )
    return f(idxt_flat, x1_flat)


def kernel(point_seq):
    b, t, n, d = point_seq.shape
    x1 = point_seq.reshape(-1, n, d)
    x1t = jnp.transpose(x1, (0, 2, 1))

    idxs_t, dists_t = _knn(x1t, x1)

    pats_t, pts_t, feats_t, norm_t = _chain(
        idxs_t.reshape(-1), x1.reshape(-1))

    def unt(a):
        return a.reshape(b, t, K, n).transpose(0, 1, 3, 2)

    def untc(a, c):
        return a.reshape(b, t, c, K, n).transpose(0, 1, 4, 3, 2)

    return (
        unt(idxs_t),
        unt(dists_t),
        unt(pats_t),
        untc(pts_t, 3),
        untc(feats_t, 6),
        untc(norm_t, 3),
    )
